# baseline (speedup 1.0000x reference)
import jax
import jax.numpy as jnp
from jax.experimental import pallas as pl

_B, _L, _D = 16, 1022, 3072
_ROWS = _B * _L
_BLKR = 584


def _fused(x_ref, mrow_ref, mask_ref, out_ref, bool_ref, ends_ref):
    out_ref[...] = x_ref[...] * mrow_ref[...]

    @pl.when(pl.program_id(0) == 0)
    def _():
        m = mask_ref[...]
        bool_ref[...] = m != 0.0
        ends_ref[...] = jnp.sum(m, axis=1, keepdims=True)


def kernel(X, attention_mask):
    x2 = X.reshape(_ROWS, _D)
    mrow = attention_mask.reshape(_ROWS, 1)
    grid = _ROWS // _BLKR
    out, mask_bool, ends = pl.pallas_call(
        _fused,
        grid=(grid,),
        in_specs=[
            pl.BlockSpec((_BLKR, _D), lambda i: (i, 0)),
            pl.BlockSpec((_BLKR, 1), lambda i: (i, 0)),
            pl.BlockSpec((_B, _L), lambda i: (0, 0)),
        ],
        out_specs=[
            pl.BlockSpec((_BLKR, _D), lambda i: (i, 0)),
            pl.BlockSpec((_B, _L), lambda i: (0, 0)),
            pl.BlockSpec((_B, 1), lambda i: (0, 0)),
        ],
        out_shape=[
            jax.ShapeDtypeStruct((_ROWS, _D), X.dtype),
            jax.ShapeDtypeStruct((_B, _L), jnp.bool_),
            jax.ShapeDtypeStruct((_B, 1), jnp.float32),
        ],
    )(x2, mrow, attention_mask)
    return out.reshape(_B, _L, _D), mask_bool, ends.reshape(_B)


# speedup vs baseline: 1.7994x; 1.7994x over previous
import jax
import jax.numpy as jnp
from jax.experimental import pallas as pl

_B, _L, _D = 16, 1022, 3072
_DBLK = 768


def _fused(x_ref, mask_ref, out_ref, bool_ref, ends_ref):
    b = pl.program_id(0)
    m = mask_ref[pl.ds(b, 1), :]
    out_ref[...] = x_ref[...] * m[:, :, None]

    @pl.when((b == 0) & (pl.program_id(1) == 0))
    def _():
        mm = mask_ref[...]
        bool_ref[...] = mm != 0.0
        ends_ref[...] = jnp.sum(mm, axis=1, keepdims=True)


def kernel(X, attention_mask):
    out, mask_bool, ends = pl.pallas_call(
        _fused,
        grid=(_B, _D // _DBLK),
        in_specs=[
            pl.BlockSpec((1, _L, _DBLK), lambda b, d: (b, 0, d)),
            pl.BlockSpec((_B, _L), lambda b, d: (0, 0)),
        ],
        out_specs=[
            pl.BlockSpec((1, _L, _DBLK), lambda b, d: (b, 0, d)),
            pl.BlockSpec((_B, _L), lambda b, d: (0, 0)),
            pl.BlockSpec((_B, 1), lambda b, d: (0, 0)),
        ],
        out_shape=[
            jax.ShapeDtypeStruct((_B, _L, _D), X.dtype),
            jax.ShapeDtypeStruct((_B, _L), jnp.bool_),
            jax.ShapeDtypeStruct((_B, 1), jnp.float32),
        ],
    )(X, attention_mask)
    return out, mask_bool, ends.reshape(_B)
